# Initial kernel scaffold; baseline (speedup 1.0000x reference)
import jax
import jax.numpy as jnp
from jax.experimental import pallas as pl


def kernel(x, qkv_w, k_fc_w, k_proj_w, v_fc_w, v_proj_w, compress_mem_kv, k_pos, v_pos, strat_w, strat_b, combine_w):
    raise NotImplementedError("write your pallas kernel here")


# speedup vs baseline: 2.6327x; 2.6327x over previous
import functools
import numpy as np
import jax
import jax.numpy as jnp
from jax.experimental import pallas as pl
from jax.experimental.pallas import tpu as pltpu

B, T, DIM = 1, 2048, 768
HEADS, DHEAD = 12, 64
HDIM = HEADS * DHEAD
CBS, SBS = 4, 4
NSEL, NMEM = 4, 1
WINDOW = 32
SCALE = 0.12
CDIM = CBS * DHEAD
HID = CDIM * 4
NBLK = T // CBS

QT = 256
NQT = T // QT
WT = 128
NWT = T // WT

NEG = -1e30


def _nt(a, b):
    return jax.lax.dot_general(a, b, (((1,), (1,)), ((), ())),
                               preferred_element_type=jnp.float32,
                               precision=jax.lax.Precision.HIGHEST)


def _nn(a, b):
    return jax.lax.dot_general(a, b, (((1,), (0,)), ((), ())),
                               preferred_element_type=jnp.float32,
                               precision=jax.lax.Precision.HIGHEST)


def _b(a):
    return a.astype(jnp.bfloat16)


def _ntd(a, b):
    return jax.lax.dot_general(_b(a), _b(b), (((1,), (1,)), ((), ())),
                               preferred_element_type=jnp.float32)


def _nnd(a, b):
    return jax.lax.dot_general(_b(a), _b(b), (((1,), (0,)), ((), ())),
                               preferred_element_type=jnp.float32)


def _rope_rot(x):
    n = x.shape[1]
    zl = pltpu.roll(x, n - 1, 1)
    zr = pltpu.roll(x, 1, 1)
    even = (jax.lax.broadcasted_iota(jnp.int32, x.shape, 1) % 2) == 0
    return jnp.where(even, -zl, zr)


def _qkv_kernel(x_ref, w_ref, c_ref, s_ref, q_ref, k_ref, v_ref):
    xt = x_ref[...]
    qkv = _ntd(xt, w_ref[...])
    q = qkv[:, :HDIM]
    k = qkv[:, HDIM:2 * HDIM]
    v = qkv[:, 2 * HDIM:]
    c = c_ref[...]
    s = s_ref[...]
    q_ref[...] = q * c + _rope_rot(q) * s
    k_ref[...] = k * c + _rope_rot(k) * s
    v_ref[...] = v


def _compress_kernel(km_ref, vm_ref, kp_ref, vp_ref,
                     kfc_ref, kpj_ref, vfc_ref, vpj_ref,
                     ck_ref, cv_ref):
    km = km_ref[0] + kp_ref[0]
    hk = _ntd(km, kfc_ref[...])
    hk = jnp.square(jnp.maximum(hk, 0.0))
    ck_ref[0] = _ntd(hk, kpj_ref[...])
    vm = vm_ref[0] + vp_ref[0]
    hv = _ntd(vm, vfc_ref[...])
    hv = jnp.square(jnp.maximum(hv, 0.0))
    cv_ref[0] = _ntd(hv, vpj_ref[...])


def _cattn_kernel(q_ref, ck_ref, cv_ref, mk_ref, mv_ref, cout_ref, imp_ref):
    i = pl.program_id(0)
    h = pl.program_id(1)
    q = q_ref[0]
    ck = ck_ref[0]
    sim = _ntd(q, ck) * SCALE
    t = i * QT + jax.lax.broadcasted_iota(jnp.int32, (QT, NBLK), 0)
    b = jax.lax.broadcasted_iota(jnp.int32, (QT, NBLK), 1)
    mask = (CBS * b + CBS - 1) < t
    sim = jnp.where(mask, sim, NEG)
    qb = _b(q).astype(jnp.float32)
    mkb = _b(mk_ref[0]).astype(jnp.float32)
    mem_logit = jnp.sum(qb * mkb, axis=-1, keepdims=True) * SCALE
    m = jnp.maximum(jnp.max(sim, axis=-1, keepdims=True), mem_logit)
    p = jnp.exp(sim - m)
    pm = jnp.exp(mem_logit - m)
    denom = jnp.sum(p, axis=-1, keepdims=True) + pm
    attn_n = p / denom
    pm_n = pm / denom
    memo = (_b(pm_n).astype(jnp.float32)) * (_b(mv_ref[0]).astype(jnp.float32))
    cout_ref[0] = _nnd(attn_n, cv_ref[0]) + memo

    @pl.when(h == 0)
    def _():
        imp_ref[...] = attn_n

    @pl.when(h > 0)
    def _():
        imp_ref[...] += attn_n


def _fine_kernel(imp_ref, q_ref, k_ref, v_ref, fout_ref, sel_ref):
    i = pl.program_id(0)
    h = pl.program_id(1)

    @pl.when(h == 0)
    def _():
        imp = imp_ref[...]
        cols = jax.lax.broadcasted_iota(jnp.int32, (QT, NBLK), 1)
        cur = imp
        for j in range(NSEL):
            m = jnp.max(cur, axis=-1, keepdims=True)
            idx = jnp.min(jnp.where(cur == m, cols, NBLK),
                          axis=-1, keepdims=True)
            sel_ref[:, j:j + 1] = idx.astype(jnp.float32)
            cur = jnp.where(cols == idx, -1.0, cur)

    q = q_ref[0]
    k = k_ref[0]
    sim = _ntd(q, k) * SCALE
    t = i * QT + jax.lax.broadcasted_iota(jnp.int32, (QT, T), 0)
    s = jax.lax.broadcasted_iota(jnp.int32, (QT, T), 1)
    sblk = s // SBS
    mask = sblk == (t // SBS)
    for j in range(NSEL):
        selj = sel_ref[:, j:j + 1].astype(jnp.int32)
        mask = mask | (sblk == selj)
    mask = mask & (s <= t)
    sim = jnp.where(mask, sim, NEG)
    m = jnp.max(sim, axis=-1, keepdims=True)
    p = jnp.exp(sim - m)
    denom = jnp.sum(p, axis=-1, keepdims=True)
    fout_ref[0] = _nnd(p / denom, v_ref[0])


def _window_kernel(q_ref, kp_ref, kc_ref, vp_ref, vc_ref, sout_ref):
    i = pl.program_id(0)
    q = q_ref[0]
    t = i * WT + jax.lax.broadcasted_iota(jnp.int32, (WT, WT), 0)
    scol = jax.lax.broadcasted_iota(jnp.int32, (WT, WT), 1)

    sp = _ntd(q, kp_ref[0]) * SCALE
    pos_p = (i - 1) * WT + scol
    mask_p = (pos_p >= 0) & (t - pos_p < WINDOW) & (pos_p <= t)
    sp = jnp.where(mask_p, sp, NEG)

    sc = _ntd(q, kc_ref[0]) * SCALE
    pos_c = i * WT + scol
    mask_c = (pos_c <= t) & (t - pos_c < WINDOW)
    sc = jnp.where(mask_c, sc, NEG)

    m = jnp.maximum(jnp.max(sp, axis=-1, keepdims=True),
                    jnp.max(sc, axis=-1, keepdims=True))
    pp = jnp.exp(sp - m)
    pc = jnp.exp(sc - m)
    denom = (jnp.sum(pp, axis=-1, keepdims=True)
             + jnp.sum(pc, axis=-1, keepdims=True))
    sout_ref[0] = _nnd(pp / denom, vp_ref[0]) + _nnd(pc / denom, vc_ref[0])


def _combine_kernel(x_ref, co_ref, fo_ref, so_ref,
                    sw_ref, sb_ref, smat_ref, cw_ref, out_ref):
    xt = x_ref[...]
    glog = _ntd(xt, sw_ref[...]) + sb_ref[...]
    g = 1.0 / (1.0 + jnp.exp(-glog))
    gates = _nn(g, smat_ref[...])
    o = (gates[:, :HDIM] * co_ref[...]
         + gates[:, HDIM:2 * HDIM] * fo_ref[...]
         + gates[:, 2 * HDIM:] * so_ref[...])
    out_ref[...] = _ntd(o, cw_ref[...])


def _rope_tables():
    inv = 1.0 / (10000.0 ** (jnp.arange(0, DHEAD, 2, dtype=jnp.float32) / DHEAD))
    freqs = jnp.arange(T, dtype=jnp.float32)[:, None] * inv[None, :]
    cos = jnp.repeat(jnp.cos(freqs), 2, axis=1)
    sin = jnp.repeat(jnp.sin(freqs), 2, axis=1)
    return jnp.tile(cos, (1, HEADS)), jnp.tile(sin, (1, HEADS))


def _gate_scatter_matrix():
    s = np.zeros((128, 3 * HDIM), dtype=np.float32)
    for h in range(HEADS):
        for j in range(3):
            s[h * 3 + j, j * HDIM + h * DHEAD: j * HDIM + (h + 1) * DHEAD] = 1.0
    return s


_SMAT = _gate_scatter_matrix()


@jax.jit
def kernel(x, qkv_w, k_fc_w, k_proj_w, v_fc_w, v_proj_w, compress_mem_kv,
           k_pos, v_pos, strat_w, strat_b, combine_w):
    f32 = jnp.float32
    x2 = x[0]
    wq = qkv_w.reshape(3 * HDIM, DIM)
    cos, sin = _rope_tables()
    smat = jnp.asarray(_SMAT)

    q, k, v = pl.pallas_call(
        _qkv_kernel,
        grid=(NQT,),
        in_specs=[
            pl.BlockSpec((QT, DIM), lambda i: (i, 0)),
            pl.BlockSpec((3 * HDIM, DIM), lambda i: (0, 0)),
            pl.BlockSpec((QT, HDIM), lambda i: (i, 0)),
            pl.BlockSpec((QT, HDIM), lambda i: (i, 0)),
        ],
        out_specs=[
            pl.BlockSpec((QT, HDIM), lambda i: (i, 0)),
            pl.BlockSpec((QT, HDIM), lambda i: (i, 0)),
            pl.BlockSpec((QT, HDIM), lambda i: (i, 0)),
        ],
        out_shape=[jax.ShapeDtypeStruct((T, HDIM), f32)] * 3,
    )(x2, wq, cos, sin)

    qh = q.reshape(T, HEADS, DHEAD).transpose(1, 0, 2)
    kh = k.reshape(T, HEADS, DHEAD).transpose(1, 0, 2)
    vh = v.reshape(T, HEADS, DHEAD).transpose(1, 0, 2)
    km = kh.reshape(HEADS, NBLK, CDIM)
    vm = vh.reshape(HEADS, NBLK, CDIM)
    kp = k_pos.reshape(HEADS, 1, CDIM)
    vp = v_pos.reshape(HEADS, 1, CDIM)

    ck, cv = pl.pallas_call(
        _compress_kernel,
        grid=(HEADS,),
        in_specs=[
            pl.BlockSpec((1, NBLK, CDIM), lambda h: (h, 0, 0)),
            pl.BlockSpec((1, NBLK, CDIM), lambda h: (h, 0, 0)),
            pl.BlockSpec((1, 1, CDIM), lambda h: (h, 0, 0)),
            pl.BlockSpec((1, 1, CDIM), lambda h: (h, 0, 0)),
            pl.BlockSpec((HID, CDIM), lambda h: (0, 0)),
            pl.BlockSpec((DHEAD, HID), lambda h: (0, 0)),
            pl.BlockSpec((HID, CDIM), lambda h: (0, 0)),
            pl.BlockSpec((DHEAD, HID), lambda h: (0, 0)),
        ],
        out_specs=[
            pl.BlockSpec((1, NBLK, DHEAD), lambda h: (h, 0, 0)),
            pl.BlockSpec((1, NBLK, DHEAD), lambda h: (h, 0, 0)),
        ],
        out_shape=[jax.ShapeDtypeStruct((HEADS, NBLK, DHEAD), f32)] * 2,
    )(km, vm, kp, vp, k_fc_w, k_proj_w, v_fc_w, v_proj_w)

    mem_k = compress_mem_kv[0].reshape(HEADS, NMEM, DHEAD)
    mem_v = compress_mem_kv[1].reshape(HEADS, NMEM, DHEAD)

    cout, imp = pl.pallas_call(
        _cattn_kernel,
        grid=(NQT, HEADS),
        in_specs=[
            pl.BlockSpec((1, QT, DHEAD), lambda i, h: (h, i, 0)),
            pl.BlockSpec((1, NBLK, DHEAD), lambda i, h: (h, 0, 0)),
            pl.BlockSpec((1, NBLK, DHEAD), lambda i, h: (h, 0, 0)),
            pl.BlockSpec((1, NMEM, DHEAD), lambda i, h: (h, 0, 0)),
            pl.BlockSpec((1, NMEM, DHEAD), lambda i, h: (h, 0, 0)),
        ],
        out_specs=[
            pl.BlockSpec((1, QT, DHEAD), lambda i, h: (h, i, 0)),
            pl.BlockSpec((QT, NBLK), lambda i, h: (i, 0)),
        ],
        out_shape=[
            jax.ShapeDtypeStruct((HEADS, T, DHEAD), f32),
            jax.ShapeDtypeStruct((T, NBLK), f32),
        ],
    )(qh, ck, cv, mem_k, mem_v)

    fout = pl.pallas_call(
        _fine_kernel,
        grid=(NQT, HEADS),
        in_specs=[
            pl.BlockSpec((QT, NBLK), lambda i, h: (i, 0)),
            pl.BlockSpec((1, QT, DHEAD), lambda i, h: (h, i, 0)),
            pl.BlockSpec((1, T, DHEAD), lambda i, h: (h, 0, 0)),
            pl.BlockSpec((1, T, DHEAD), lambda i, h: (h, 0, 0)),
        ],
        out_specs=pl.BlockSpec((1, QT, DHEAD), lambda i, h: (h, i, 0)),
        out_shape=jax.ShapeDtypeStruct((HEADS, T, DHEAD), f32),
        scratch_shapes=[pltpu.VMEM((QT, 128), f32)],
    )(imp, qh, kh, vh)

    sout = pl.pallas_call(
        _window_kernel,
        grid=(NWT, HEADS),
        in_specs=[
            pl.BlockSpec((1, WT, DHEAD), lambda i, h: (h, i, 0)),
            pl.BlockSpec((1, WT, DHEAD),
                         lambda i, h: (h, jnp.maximum(i - 1, 0), 0)),
            pl.BlockSpec((1, WT, DHEAD), lambda i, h: (h, i, 0)),
            pl.BlockSpec((1, WT, DHEAD),
                         lambda i, h: (h, jnp.maximum(i - 1, 0), 0)),
            pl.BlockSpec((1, WT, DHEAD), lambda i, h: (h, i, 0)),
        ],
        out_specs=pl.BlockSpec((1, WT, DHEAD), lambda i, h: (h, i, 0)),
        out_shape=jax.ShapeDtypeStruct((HEADS, T, DHEAD), f32),
    )(qh, kh, kh, vh, vh)

    cout = cout.transpose(1, 0, 2).reshape(T, HDIM)
    fout = fout.transpose(1, 0, 2).reshape(T, HDIM)
    sout = sout.transpose(1, 0, 2).reshape(T, HDIM)

    sw = jnp.zeros((128, DIM), f32).at[:3 * HEADS].set(strat_w)
    sb = jnp.zeros((1, 128), f32).at[0, :3 * HEADS].set(strat_b)
    out = pl.pallas_call(
        _combine_kernel,
        grid=(NQT,),
        in_specs=[
            pl.BlockSpec((QT, DIM), lambda i: (i, 0)),
            pl.BlockSpec((QT, HDIM), lambda i: (i, 0)),
            pl.BlockSpec((QT, HDIM), lambda i: (i, 0)),
            pl.BlockSpec((QT, HDIM), lambda i: (i, 0)),
            pl.BlockSpec((128, DIM), lambda i: (0, 0)),
            pl.BlockSpec((1, 128), lambda i: (0, 0)),
            pl.BlockSpec((128, 3 * HDIM), lambda i: (0, 0)),
            pl.BlockSpec((DIM, HDIM), lambda i: (0, 0)),
        ],
        out_specs=pl.BlockSpec((QT, DIM), lambda i: (i, 0)),
        out_shape=jax.ShapeDtypeStruct((T, DIM), f32),
    )(x2, cout, fout, sout, sw, sb, smat, combine_w)

    return out[None]
